# Initial kernel scaffold; baseline (speedup 1.0000x reference)
import jax
import jax.numpy as jnp
from jax.experimental import pallas as pl


def kernel(x, edge_index, t, batch, W_atom, b_atom, Wg, bg, gamma, beta, W_op, b_op, Wt1, bt1, Wt2, bt2, Wn1, bn1, Wn2, bn2, Wn3, bn3):
    raise NotImplementedError("write your pallas kernel here")


# speedup vs baseline: 8.3407x; 8.3407x over previous
import functools
import math

import jax
import jax.numpy as jnp
from jax import lax
from jax.experimental import pallas as pl
from jax.experimental.pallas import tpu as pltpu
from jax.experimental.pallas import tpu_sc as plsc

N = 10000
H = 128
E = 320000
EN = E + N
NT = 16
NC = 2
NW = NT * NC
NPAD = 10240
RB = NPAD // NT
K = 128
C = 2
SUP = 41
EPT = SUP * C * K
EPAD = NW * EPT
BR = 400
G = N // BR
BNSCALE = 1.0 / math.sqrt(1.0 + 1e-5)

_mesh = plsc.VectorSubcoreMesh(core_axis_name="c", subcore_axis_name="s")


@functools.partial(
    pl.kernel,
    out_type=jax.ShapeDtypeStruct((NC * NPAD,), jnp.float32),
    mesh=_mesh,
    scratch_types=[
        pltpu.VMEM((C, K), jnp.int32),
        pltpu.VMEM((K,), jnp.float32),
        pltpu.VMEM((RB,), jnp.float32),
        pltpu.VMEM_SHARED((NPAD,), jnp.float32),
    ],
)
def _deg_kernel(dst_hbm, ones_hbm, zeros_hbm, out_hbm, didx, ones_v, buf, acc):
    c = lax.axis_index("c")
    s = lax.axis_index("s")
    w = c * NT + s
    pltpu.sync_copy(ones_hbm, ones_v)
    pltpu.sync_copy(zeros_hbm, buf)
    pltpu.sync_copy(buf, acc.at[pl.ds(s * RB, RB)])
    plsc.subcore_barrier()
    row0 = w * (SUP * C)

    def body(g, carry):
        base = row0 + g * C
        pltpu.sync_copy(dst_hbm.at[pl.ds(base, C)], didx)
        for j in range(C):
            pltpu.sync_copy(ones_v, acc.at[didx.at[j]], add=True)
        return carry

    lax.fori_loop(0, SUP, body, 0)
    plsc.subcore_barrier()
    pltpu.sync_copy(acc.at[pl.ds(s * RB, RB)], buf)
    pltpu.sync_copy(buf, out_hbm.at[pl.ds(c * NPAD + s * RB, RB)])


@functools.partial(
    pl.kernel,
    out_type=jax.ShapeDtypeStruct((NC * NPAD, H), jnp.float32),
    mesh=_mesh,
    scratch_types=[
        pltpu.VMEM((C, K), jnp.int32),
        pltpu.VMEM((C, K), jnp.int32),
        pltpu.VMEM((C * K, H), jnp.float32),
        pltpu.VMEM_SHARED((NPAD, H), jnp.float32),
        pltpu.SemaphoreType.DMA,
    ],
)
def _agg_kernel(u_hbm, src_hbm, dst_hbm, zeros_hbm, out_hbm,
                sidx, didx, rows, acc, sem):
    c = lax.axis_index("c")
    s = lax.axis_index("s")
    w = c * NT + s
    pltpu.sync_copy(zeros_hbm, rows.at[pl.ds(0, K)])
    for r in range(RB // K):
        pltpu.sync_copy(rows.at[pl.ds(0, K)], acc.at[pl.ds(s * RB + r * K, K)])
    plsc.subcore_barrier()
    row0 = w * (SUP * C)

    def body(g, carry):
        base = row0 + g * C
        pltpu.sync_copy(src_hbm.at[pl.ds(base, C)], sidx)
        pltpu.sync_copy(dst_hbm.at[pl.ds(base, C)], didx)
        descs = [
            pltpu.async_copy(u_hbm.at[sidx.at[j]], rows.at[pl.ds(j * K, K)], sem)
            for j in range(C)
        ]
        for d in descs:
            d.wait()
        for j in range(C):
            pltpu.sync_copy(rows.at[pl.ds(j * K, K)], acc.at[didx.at[j]], add=True)
        return carry

    lax.fori_loop(0, SUP, body, 0)
    plsc.subcore_barrier()
    for r in range(RB // K):
        pltpu.sync_copy(acc.at[pl.ds(s * RB + r * K, K)], rows.at[pl.ds(0, K)])
        pltpu.sync_copy(rows.at[pl.ds(0, K)],
                        out_hbm.at[pl.ds(c * NPAD + s * RB + r * K, K)])


def _row_spec():
    return pl.BlockSpec((BR, H), lambda i: (i, 0))


def _w_spec():
    return pl.BlockSpec((H, H), lambda i: (0, 0))


def _b_spec():
    return pl.BlockSpec((1, H), lambda i: (0, 0))


def _sigmoid(v):
    return 1.0 / (1.0 + jnp.exp(-v))


def _t0_body(x_ref, wa_ref, ba_ref, wg0_ref, dinv_ref, h0_ref, u1_ref):
    h0 = jnp.dot(x_ref[...], wa_ref[...], preferred_element_type=jnp.float32)
    h0 = h0 + ba_ref[...]
    h0_ref[...] = h0
    u1 = jnp.dot(h0, wg0_ref[...], preferred_element_type=jnp.float32)
    u1_ref[...] = u1 * dinv_ref[...]


_t0_call = pl.pallas_call(
    _t0_body,
    grid=(G,),
    in_specs=[_row_spec(), _w_spec(), _b_spec(), _w_spec(), _row_spec()],
    out_specs=[_row_spec(), _row_spec()],
    out_shape=[
        jax.ShapeDtypeStruct((N, H), jnp.float32),
        jax.ShapeDtypeStruct((N, H), jnp.float32),
    ],
)


def _mid_body(p0_ref, p1_ref, hp_ref, dinv_ref, bg_ref, ga_ref, be_ref,
              wgn_ref, h_ref, u_ref):
    agg = p0_ref[...] + p1_ref[...]
    hh = dinv_ref[...] * agg + bg_ref[...]
    hh = ga_ref[...] * (hh * BNSCALE) + be_ref[...]
    h = jnp.maximum(hh, 0.0) + hp_ref[...]
    h_ref[...] = h
    u = jnp.dot(h, wgn_ref[...], preferred_element_type=jnp.float32)
    u_ref[...] = u * dinv_ref[...]


_mid_call = pl.pallas_call(
    _mid_body,
    grid=(G,),
    in_specs=[_row_spec(), _row_spec(), _row_spec(), _row_spec(),
              _b_spec(), _b_spec(), _b_spec(), _w_spec()],
    out_specs=[_row_spec(), _row_spec()],
    out_shape=[
        jax.ShapeDtypeStruct((N, H), jnp.float32),
        jax.ShapeDtypeStruct((N, H), jnp.float32),
    ],
)


def _te_body(tb_ref, wt1_ref, bt1_ref, wt2_ref, bt2_ref, wn1l_ref, te2_ref):
    j = lax.broadcasted_iota(jnp.int32, (64, 64), 1).astype(jnp.float32)
    freq = jnp.exp(j * (-math.log(10000.0) / 63.0))
    arg = tb_ref[...] * freq
    emb = jnp.concatenate([jnp.sin(arg), jnp.cos(arg)], axis=1)
    v = jnp.dot(emb, wt1_ref[...], preferred_element_type=jnp.float32)
    v = v + bt1_ref[...]
    v = v * _sigmoid(v)
    v = jnp.dot(v, wt2_ref[...], preferred_element_type=jnp.float32)
    v = v + bt2_ref[...]
    te2_ref[...] = jnp.dot(v, wn1l_ref[...], preferred_element_type=jnp.float32)


_te_call = pl.pallas_call(
    _te_body,
    grid=(1,),
    in_specs=[pl.BlockSpec((64, 64), lambda i: (0, 0)), _w_spec(), _b_spec(),
              _w_spec(), _b_spec(), _w_spec()],
    out_specs=pl.BlockSpec((64, H), lambda i: (0, 0)),
    out_shape=jax.ShapeDtypeStruct((64, H), jnp.float32),
)


def _fin_body(p0_ref, p1_ref, hp_ref, dinv_ref, bg_ref, ga_ref, be_ref,
              wop_ref, bop_ref, bb_ref, te2_ref, wn1u_ref, bn1_ref,
              wn2_ref, bn2_ref, wn3_ref, bn3_ref, out_ref):
    agg = p0_ref[...] + p1_ref[...]
    hh = dinv_ref[...] * agg + bg_ref[...]
    hh = ga_ref[...] * (hh * BNSCALE) + be_ref[...]
    h3 = jnp.maximum(hh, 0.0) + hp_ref[...]
    hn = jnp.dot(h3, wop_ref[...], preferred_element_type=jnp.float32)
    hn = hn + bop_ref[...]
    ids = lax.broadcasted_iota(jnp.int32, (BR, 64), 1).astype(jnp.float32)
    oh = (bb_ref[...] == ids).astype(jnp.float32)
    tn = jnp.dot(oh, te2_ref[...], preferred_element_type=jnp.float32)
    a = jnp.dot(hn, wn1u_ref[...], preferred_element_type=jnp.float32)
    a = a + tn + bn1_ref[...]
    a = a * _sigmoid(a)
    b = jnp.dot(a, wn2_ref[...], preferred_element_type=jnp.float32)
    b = b + bn2_ref[...]
    b = b * _sigmoid(b)
    o = jnp.dot(b, wn3_ref[...], preferred_element_type=jnp.float32)
    out_ref[...] = o + bn3_ref[...]


_fin_call = pl.pallas_call(
    _fin_body,
    grid=(G,),
    in_specs=[_row_spec(), _row_spec(), _row_spec(), _row_spec(),
              _b_spec(), _b_spec(), _b_spec(),
              _w_spec(), _b_spec(),
              pl.BlockSpec((BR, 64), lambda i: (i, 0)),
              pl.BlockSpec((64, H), lambda i: (0, 0)),
              _w_spec(), _b_spec(), _w_spec(), _b_spec(), _w_spec(), _b_spec()],
    out_specs=_row_spec(),
    out_shape=jax.ShapeDtypeStruct((N, H), jnp.float32),
)


def kernel(x, edge_index, t, batch, W_atom, b_atom, Wg, bg, gamma, beta,
           W_op, b_op, Wt1, bt1, Wt2, bt2, Wn1, bn1, Wn2, bn2, Wn3, bn3):
    f32 = jnp.float32
    si = jnp.arange(N, dtype=jnp.int32)
    pad = EPAD - EN
    src = jnp.concatenate(
        [edge_index[0].astype(jnp.int32), si, jnp.zeros((pad,), jnp.int32)])
    dst = jnp.concatenate(
        [edge_index[1].astype(jnp.int32), si, jnp.full((pad,), N, jnp.int32)])
    src2 = src.reshape(EPAD // K, K)
    dst2 = dst.reshape(EPAD // K, K)
    zrows = jnp.zeros((K, H), f32)
    ones_k = jnp.ones((K,), f32)
    zrb = jnp.zeros((RB,), f32)

    degp = _deg_kernel(dst2, ones_k, zrb)
    deg = degp[:NPAD][:N] + degp[NPAD:][:N]
    dinv = jnp.where(deg > 0, lax.rsqrt(deg), 0.0)
    dinvb = jnp.broadcast_to(dinv[:, None], (N, H))

    ba2 = b_atom.reshape(1, H)
    h0, u1 = _t0_call(x, W_atom, ba2, Wg[0], dinvb)

    hprev = h0
    u = u1
    for i in range(2):
        aggp = _agg_kernel(u, src2, dst2, zrows)
        hprev, u = _mid_call(aggp[:N], aggp[NPAD:NPAD + N], hprev, dinvb,
                             bg[i].reshape(1, H), gamma[i].reshape(1, H),
                             beta[i].reshape(1, H), Wg[i + 1])

    aggp = _agg_kernel(u, src2, dst2, zrows)

    tb = jnp.broadcast_to(t.astype(f32)[:, None], (64, 64))
    te2 = _te_call(tb, Wt1, bt1.reshape(1, H), Wt2, bt2.reshape(1, H), Wn1[H:])

    batchb = jnp.broadcast_to(batch.astype(f32)[:, None], (N, 64))
    out = _fin_call(aggp[:N], aggp[NPAD:NPAD + N], hprev, dinvb,
                    bg[2].reshape(1, H), gamma[2].reshape(1, H),
                    beta[2].reshape(1, H), W_op, b_op.reshape(1, H),
                    batchb, te2, Wn1[:H], bn1.reshape(1, H),
                    Wn2, bn2.reshape(1, H), Wn3, bn3.reshape(1, H))
    return out
